# baseline (device time: 370084 ns/iter reference)
import jax
import jax.numpy as jnp
from jax import lax
from jax.experimental import pallas as pl
from jax.experimental.pallas import tpu as pltpu

N_DEV = 8
N_EXPERTS = 32
E_PER = 4
CAP = 204


def _hist_allgather(hist):

    def body(hist_ref, out_ref, send_sems, recv_sems):
        my = lax.axis_index("i")

        barrier = pltpu.get_barrier_semaphore()
        for d in range(1, N_DEV):
            pl.semaphore_signal(
                barrier, inc=1,
                device_id=((my + d) % N_DEV,),
                device_id_type=pl.DeviceIdType.MESH,
            )
        pl.semaphore_wait(barrier, N_DEV - 1)

        out_ref[pl.ds(my, 1)] = hist_ref[...]

        rdmas = []
        for d in range(1, N_DEV):
            rdma = pltpu.make_async_remote_copy(
                src_ref=hist_ref,
                dst_ref=out_ref.at[pl.ds(my, 1)],
                send_sem=send_sems.at[d],
                recv_sem=recv_sems.at[my],
                device_id=((my + d) % N_DEV,),
                device_id_type=pl.DeviceIdType.MESH,
            )
            rdma.start()
            rdmas.append(rdma)
        for rdma in rdmas:
            rdma.wait_send()

        for s in range(N_DEV):
            @pl.when(s != my)
            def _(s=s):
                recv = pltpu.make_async_remote_copy(
                    src_ref=hist_ref,
                    dst_ref=out_ref.at[pl.ds(s, 1)],
                    send_sem=send_sems.at[0],
                    recv_sem=recv_sems.at[s],
                    device_id=(0,),
                    device_id_type=pl.DeviceIdType.MESH,
                )
                recv.wait_recv()

    return pl.pallas_call(
        body,
        out_shape=jax.ShapeDtypeStruct((N_DEV, 128), jnp.int32),
        in_specs=[pl.BlockSpec(memory_space=pltpu.VMEM)],
        out_specs=pl.BlockSpec(memory_space=pltpu.VMEM),
        scratch_shapes=[
            pltpu.SemaphoreType.DMA((N_DEV,)),
            pltpu.SemaphoreType.DMA((N_DEV,)),
        ],
        compiler_params=pltpu.CompilerParams(collective_id=0),
    )(hist)


def _moe_ring(x_bf, a_perm, w_bf):
    n_tok, d_in = x_bf.shape
    d_out = w_bf.shape[-1]

    def body(x_ref, a_ref, w_ref, out_ref, wbuf, send_sems, recv_sems,
             credit_sem):
        my = lax.axis_index("i")
        left = (my - 1) % N_DEV
        right = (my + 1) % N_DEV

        barrier = pltpu.get_barrier_semaphore()
        for nbr in (left, right):
            pl.semaphore_signal(
                barrier, inc=1,
                device_id=(nbr,), device_id_type=pl.DeviceIdType.MESH,
            )
        pl.semaphore_wait(barrier, 2)

        wbuf[0] = w_ref[...]

        for h in range(N_DEV):
            cur = h % 2
            rdma = None
            if h < N_DEV - 1:
                if h >= 1:
                    pl.semaphore_wait(credit_sem, 1)
                rdma = pltpu.make_async_remote_copy(
                    src_ref=wbuf.at[cur],
                    dst_ref=wbuf.at[1 - cur],
                    send_sem=send_sems.at[cur],
                    recv_sem=recv_sems.at[1 - cur],
                    device_id=(right,),
                    device_id_type=pl.DeviceIdType.MESH,
                )
                rdma.start()

            for j in range(E_PER):
                col = h * E_PER + j
                xm = x_ref[...] * a_ref[:, col:col + 1]
                contrib = jnp.dot(
                    xm, wbuf[cur, j], preferred_element_type=jnp.float32
                )
                if h == 0 and j == 0:
                    out_ref[...] = contrib
                else:
                    out_ref[...] += contrib

            if h < N_DEV - 1:
                rdma.wait()
                if h < N_DEV - 2:
                    pl.semaphore_signal(
                        credit_sem, inc=1,
                        device_id=(left,),
                        device_id_type=pl.DeviceIdType.MESH,
                    )

    return pl.pallas_call(
        body,
        out_shape=jax.ShapeDtypeStruct((n_tok, d_out), jnp.float32),
        in_specs=[pl.BlockSpec(memory_space=pltpu.VMEM)] * 3,
        out_specs=pl.BlockSpec(memory_space=pltpu.VMEM),
        scratch_shapes=[
            pltpu.VMEM((2, E_PER, d_in, d_out), jnp.bfloat16),
            pltpu.SemaphoreType.DMA((2,)),
            pltpu.SemaphoreType.DMA((2,)),
            pltpu.SemaphoreType.REGULAR,
        ],
        compiler_params=pltpu.CompilerParams(collective_id=1),
    )(x_bf, a_perm, w_bf)


def kernel(x, router_W, route_idx, expert_W):
    del router_W
    my = lax.axis_index("i")

    e = route_idx[:, 0]
    onehot = (e[:, None] == jnp.arange(N_EXPERTS, dtype=e.dtype)[None, :])
    onehot = onehot.astype(jnp.int32)
    hist = jnp.sum(onehot, axis=0)
    hist_pad = jnp.zeros((1, 128), jnp.int32).at[0, :N_EXPERTS].set(hist)

    all_hist = _hist_allgather(hist_pad)[:, :N_EXPERTS]

    before = (jnp.arange(N_DEV) < my)[:, None]
    prefix = jnp.sum(jnp.where(before, all_hist, 0), axis=0)
    excl_rank = jnp.cumsum(onehot, axis=0) - onehot
    accept = (onehot > 0) & ((prefix[None, :] + excl_rank) < CAP)

    owners = (my - jnp.arange(N_DEV)) % N_DEV
    perm = (owners[:, None] * E_PER + jnp.arange(E_PER)[None, :]).reshape(-1)
    a_perm = jnp.take(accept.astype(jnp.bfloat16), perm, axis=1)

    return _moe_ring(
        x.astype(jnp.bfloat16), a_perm, expert_W.astype(jnp.bfloat16)
    )


# device time: 214261 ns/iter; 1.7273x vs baseline; 1.7273x over previous
import jax
import jax.numpy as jnp
from jax import lax
from jax.experimental import pallas as pl
from jax.experimental.pallas import tpu as pltpu

N_DEV = 8
N_EXPERTS = 32
E_PER = 4
CAP = 204


def _hist_allgather(hist):

    def body(hist_ref, out_ref, send_sems, recv_sems):
        my = lax.axis_index("i")

        barrier = pltpu.get_barrier_semaphore()
        for d in range(1, N_DEV):
            pl.semaphore_signal(
                barrier, inc=1,
                device_id=((my + d) % N_DEV,),
                device_id_type=pl.DeviceIdType.MESH,
            )
        pl.semaphore_wait(barrier, N_DEV - 1)

        out_ref[pl.ds(my, 1)] = hist_ref[...]

        rdmas = []
        for d in range(1, N_DEV):
            rdma = pltpu.make_async_remote_copy(
                src_ref=hist_ref,
                dst_ref=out_ref.at[pl.ds(my, 1)],
                send_sem=send_sems.at[d],
                recv_sem=recv_sems.at[my],
                device_id=((my + d) % N_DEV,),
                device_id_type=pl.DeviceIdType.MESH,
            )
            rdma.start()
            rdmas.append(rdma)
        for rdma in rdmas:
            rdma.wait_send()

        for s in range(N_DEV):
            @pl.when(s != my)
            def _(s=s):
                recv = pltpu.make_async_remote_copy(
                    src_ref=hist_ref,
                    dst_ref=out_ref.at[pl.ds(s, 1)],
                    send_sem=send_sems.at[0],
                    recv_sem=recv_sems.at[s],
                    device_id=(0,),
                    device_id_type=pl.DeviceIdType.MESH,
                )
                recv.wait_recv()

    return pl.pallas_call(
        body,
        out_shape=jax.ShapeDtypeStruct((N_DEV, 128), jnp.int32),
        in_specs=[pl.BlockSpec(memory_space=pltpu.VMEM)],
        out_specs=pl.BlockSpec(memory_space=pltpu.VMEM),
        scratch_shapes=[
            pltpu.SemaphoreType.DMA((N_DEV,)),
            pltpu.SemaphoreType.DMA((N_DEV,)),
        ],
        compiler_params=pltpu.CompilerParams(collective_id=0),
    )(hist)


def _moe_ring(x_bf, a_perm, w_cw, w_ccw):
    n_tok, d_in = x_bf.shape
    d_out = w_cw.shape[-1]
    k2 = 2 * d_in

    def body(x_ref, a_ref, wcw_ref, wccw_ref, out_ref, bcw, bccw,
             ssem_cw, rsem_cw, ssem_ccw, rsem_ccw, cred_cw, cred_ccw):
        my = lax.axis_index("i")
        left = (my - 1) % N_DEV
        right = (my + 1) % N_DEV

        barrier = pltpu.get_barrier_semaphore()
        for nbr in (left, right):
            pl.semaphore_signal(
                barrier, inc=1,
                device_id=(nbr,), device_id_type=pl.DeviceIdType.MESH,
            )
        pl.semaphore_wait(barrier, 2)

        bcw[0] = wcw_ref[...]
        bccw[0] = wccw_ref[...]

        for h in range(N_DEV):
            cur = h % 2
            rcw = rccw = None
            if h < N_DEV - 1:
                if h >= 1:
                    pl.semaphore_wait(cred_cw, 1)
                    pl.semaphore_wait(cred_ccw, 1)
                rcw = pltpu.make_async_remote_copy(
                    src_ref=bcw.at[cur],
                    dst_ref=bcw.at[1 - cur],
                    send_sem=ssem_cw.at[cur],
                    recv_sem=rsem_cw.at[1 - cur],
                    device_id=(right,),
                    device_id_type=pl.DeviceIdType.MESH,
                )
                rccw = pltpu.make_async_remote_copy(
                    src_ref=bccw.at[cur],
                    dst_ref=bccw.at[1 - cur],
                    send_sem=ssem_ccw.at[cur],
                    recv_sem=rsem_ccw.at[1 - cur],
                    device_id=(left,),
                    device_id_type=pl.DeviceIdType.MESH,
                )
                rcw.start()
                rccw.start()

            c = h * E_PER
            x_all = x_ref[...]
            xm_cw = jnp.concatenate(
                [x_all * a_ref[:, c:c + 1], x_all * a_ref[:, c + 1:c + 2]],
                axis=1,
            )
            acc = jnp.dot(xm_cw, bcw[cur], preferred_element_type=jnp.float32)
            xm_ccw = jnp.concatenate(
                [x_all * a_ref[:, c + 2:c + 3], x_all * a_ref[:, c + 3:c + 4]],
                axis=1,
            )
            acc = acc + jnp.dot(
                xm_ccw, bccw[cur], preferred_element_type=jnp.float32
            )
            if h == 0:
                out_ref[...] = acc
            else:
                out_ref[...] += acc

            if h < N_DEV - 1:
                rcw.wait()
                rccw.wait()
                if h < N_DEV - 2:
                    pl.semaphore_signal(
                        cred_cw, inc=1,
                        device_id=(left,),
                        device_id_type=pl.DeviceIdType.MESH,
                    )
                    pl.semaphore_signal(
                        cred_ccw, inc=1,
                        device_id=(right,),
                        device_id_type=pl.DeviceIdType.MESH,
                    )

    return pl.pallas_call(
        body,
        out_shape=jax.ShapeDtypeStruct((n_tok, d_out), jnp.float32),
        in_specs=[pl.BlockSpec(memory_space=pltpu.VMEM)] * 4,
        out_specs=pl.BlockSpec(memory_space=pltpu.VMEM),
        scratch_shapes=[
            pltpu.VMEM((2, k2, d_out), jnp.bfloat16),
            pltpu.VMEM((2, k2, d_out), jnp.bfloat16),
            pltpu.SemaphoreType.DMA((2,)),
            pltpu.SemaphoreType.DMA((2,)),
            pltpu.SemaphoreType.DMA((2,)),
            pltpu.SemaphoreType.DMA((2,)),
            pltpu.SemaphoreType.REGULAR,
            pltpu.SemaphoreType.REGULAR,
        ],
        compiler_params=pltpu.CompilerParams(collective_id=1),
    )(x_bf, a_perm, w_cw, w_ccw)


def kernel(x, router_W, route_idx, expert_W):
    del router_W
    my = lax.axis_index("i")

    e = route_idx[:, 0]
    onehot = (e[:, None] == jnp.arange(N_EXPERTS, dtype=e.dtype)[None, :])
    onehot = onehot.astype(jnp.int32)
    hist = jnp.sum(onehot, axis=0)
    hist_pad = jnp.zeros((1, 128), jnp.int32).at[0, :N_EXPERTS].set(hist)

    all_hist = _hist_allgather(hist_pad)[:, :N_EXPERTS]

    before = (jnp.arange(N_DEV) < my)[:, None]
    prefix = jnp.sum(jnp.where(before, all_hist, 0), axis=0)
    excl_rank = jnp.cumsum(onehot, axis=0) - onehot
    accept = (onehot > 0) & ((prefix[None, :] + excl_rank) < CAP)

    owners_cw = (my - jnp.arange(N_DEV)) % N_DEV
    owners_ccw = (my + jnp.arange(N_DEV)) % N_DEV
    perm = jnp.stack(
        [owners_cw * E_PER, owners_cw * E_PER + 1,
         owners_ccw * E_PER + 2, owners_ccw * E_PER + 3],
        axis=1,
    ).reshape(-1)
    a_perm = jnp.take(accept.astype(jnp.bfloat16), perm, axis=1)

    w = expert_W.astype(jnp.bfloat16)
    w_cw = w[0:2].reshape(2 * w.shape[1], w.shape[2])
    w_ccw = w[2:4].reshape(2 * w.shape[1], w.shape[2])

    return _moe_ring(x.astype(jnp.bfloat16), a_perm, w_cw, w_ccw)
